# baseline (speedup 1.0000x reference)
import functools

import jax
import jax.numpy as jnp
from jax import lax
from jax.experimental import pallas as pl
from jax.experimental.pallas import tpu as pltpu
from jax.experimental.pallas import tpu_sc as plsc

VOCAB = 100000
N_EMBD = 1024
B = 4
T = 2048

NC = 2
NS = 16
NW = NC * NS

TW = T // NW
C = 16
K = 5
NTC = TW // C
NCHUNK = B * NTC
GROUPS = C * N_EMBD // 16


def _body(idx_hbm, tok_hbm, pos_hbm, out_hbm, idx_v, pos0, pos1, *scratch):
    rows = list(scratch[:K])
    sem_g = list(scratch[K:2 * K])
    sem_s = list(scratch[2 * K:3 * K])
    sem_ix = scratch[3 * K]
    sem_pos = [scratch[3 * K + 1], scratch[3 * K + 2]]
    pos = [pos0, pos1]

    wid = lax.axis_index("s") * NC + lax.axis_index("c")
    t0 = wid * TW

    def issue_gather(q):
        tc, b = divmod(q, B)
        return pltpu.async_copy(
            tok_hbm.at[idx_v.at[b, pl.ds(tc * C, C)]], rows[q % K],
            sem_g[q % K])

    def issue_store(q):
        tc, b = divmod(q, B)
        return pltpu.async_copy(
            rows[q % K], out_hbm.at[b, pl.ds(t0 + tc * C, C)], sem_s[q % K])

    def issue_pos(tc):
        return pltpu.async_copy(
            pos_hbm.at[0, pl.ds(t0 + tc * C, C)], pos[tc % 2],
            sem_pos[tc % 2])

    def vpu_add(q):
        tc = q // B
        buf = rows[q % K]
        pbuf = pos[tc % 2]

        @plsc.parallel_loop(0, GROUPS, unroll=16)
        def _(g):
            r = g >> 6
            c = (g & 63) * 16
            plsc.addupdate(buf.at[r, pl.ds(c, 16)], pbuf[r, pl.ds(c, 16)])

    ix = [pltpu.async_copy(idx_hbm.at[b, pl.ds(t0, TW)],
                           idx_v.at[b], sem_ix) for b in range(B)]
    P = [None] * NTC
    P[0] = issue_pos(0)
    P[1] = issue_pos(1)
    for d in ix:
        d.wait()
    G = [None] * NCHUNK
    S = [None] * NCHUNK
    for i in range(K - 1):
        G[i] = issue_gather(i)
    for q in range(NCHUNK):
        tc, b = divmod(q, B)
        G[q].wait()
        if b == 0:
            P[tc].wait()
        vpu_add(q)
        S[q] = issue_store(q)
        if b == B - 1 and tc + 2 < NTC:
            P[tc + 2] = issue_pos(tc + 2)
        if q + K - 1 < NCHUNK:
            if q - 1 >= 0:
                S[q - 1].wait()
            G[q + K - 1] = issue_gather(q + K - 1)
    for q in range(NCHUNK - K, NCHUNK):
        S[q].wait()


_mesh = plsc.VectorSubcoreMesh(core_axis_name="c", subcore_axis_name="s")

_sc_call = functools.partial(
    pl.kernel,
    out_type=jax.ShapeDtypeStruct((B, T, N_EMBD), jnp.float32),
    mesh=_mesh,
    scratch_types=[
        pltpu.VMEM((B, TW), jnp.int32),
        pltpu.VMEM((C, N_EMBD), jnp.float32),
        pltpu.VMEM((C, N_EMBD), jnp.float32),
    ]
    + [pltpu.VMEM((C, N_EMBD), jnp.float32)] * K
    + [pltpu.SemaphoreType.DMA] * (2 * K + 3),
)(_body)


@jax.jit
def kernel(idx, tok_emb, pos_emb):
    return _sc_call(idx.astype(jnp.int32), tok_emb, pos_emb)


# speedup vs baseline: 1.0049x; 1.0049x over previous
import functools

import jax
import jax.numpy as jnp
from jax import lax
from jax.experimental import pallas as pl
from jax.experimental.pallas import tpu as pltpu
from jax.experimental.pallas import tpu_sc as plsc

VOCAB = 100000
N_EMBD = 1024
B = 4
T = 2048

NC = 2
NS = 16
NW = NC * NS

TW = T // NW
C = 16
K = 4
NTC = TW // C
NCHUNK = B * NTC
GROUPS = C * N_EMBD // 16


def _body(idx_hbm, tok_hbm, pos_hbm, out_hbm, idx_v, pos0, pos1, *scratch):
    rows = list(scratch[:K])
    sem_g = list(scratch[K:2 * K])
    sem_s = list(scratch[2 * K:3 * K])
    sem_ix = scratch[3 * K]
    sem_pos = [scratch[3 * K + 1], scratch[3 * K + 2]]
    pos = [pos0, pos1]

    wid = lax.axis_index("s") * NC + lax.axis_index("c")
    t0 = wid * TW

    def issue_gather(q):
        tc, b = divmod(q, B)
        return pltpu.async_copy(
            tok_hbm.at[idx_v.at[b, pl.ds(tc * C, C)]], rows[q % K],
            sem_g[q % K])

    def issue_store(q):
        tc, b = divmod(q, B)
        return pltpu.async_copy(
            rows[q % K], out_hbm.at[b, pl.ds(t0 + tc * C, C)], sem_s[q % K])

    def issue_pos(tc):
        return pltpu.async_copy(
            pos_hbm.at[0, pl.ds(t0 + tc * C, C)], pos[tc % 2],
            sem_pos[tc % 2])

    def vpu_add(q):
        tc = q // B
        buf = rows[q % K]
        pbuf = pos[tc % 2]

        @plsc.parallel_loop(0, GROUPS, unroll=16)
        def _(g):
            r = g >> 6
            c = (g & 63) * 16
            plsc.addupdate(buf.at[r, pl.ds(c, 16)], pbuf[r, pl.ds(c, 16)])

    ix = [pltpu.async_copy(idx_hbm.at[b, pl.ds(t0, TW)],
                           idx_v.at[b], sem_ix) for b in range(B)]
    P = [None] * NTC
    P[0] = issue_pos(0)
    P[1] = issue_pos(1)
    for d in ix:
        d.wait()
    G = [None] * NCHUNK
    S = [None] * NCHUNK
    for i in range(K - 1):
        G[i] = issue_gather(i)
    for q in range(NCHUNK):
        tc, b = divmod(q, B)
        G[q].wait()
        if b == 0:
            P[tc].wait()
        vpu_add(q)
        S[q] = issue_store(q)
        if b == B - 1 and tc + 2 < NTC:
            P[tc + 2] = issue_pos(tc + 2)
        if q + K - 1 < NCHUNK:
            if q - 1 >= 0:
                S[q - 1].wait()
            G[q + K - 1] = issue_gather(q + K - 1)
    for q in range(NCHUNK - K, NCHUNK):
        S[q].wait()


_mesh = plsc.VectorSubcoreMesh(core_axis_name="c", subcore_axis_name="s")

_sc_call = functools.partial(
    pl.kernel,
    out_type=jax.ShapeDtypeStruct((B, T, N_EMBD), jnp.float32),
    mesh=_mesh,
    scratch_types=[
        pltpu.VMEM((B, TW), jnp.int32),
        pltpu.VMEM((C, N_EMBD), jnp.float32),
        pltpu.VMEM((C, N_EMBD), jnp.float32),
    ]
    + [pltpu.VMEM((C, N_EMBD), jnp.float32)] * K
    + [pltpu.SemaphoreType.DMA] * (2 * K + 3),
)(_body)


@jax.jit
def kernel(idx, tok_emb, pos_emb):
    return _sc_call(idx.astype(jnp.int32), tok_emb, pos_emb)


# speedup vs baseline: 1.0469x; 1.0418x over previous
import functools

import jax
import jax.numpy as jnp
from jax import lax
from jax.experimental import pallas as pl
from jax.experimental.pallas import tpu as pltpu
from jax.experimental.pallas import tpu_sc as plsc

VOCAB = 100000
N_EMBD = 1024
B = 4
T = 2048

NC = 2
NS = 16
NW = NC * NS

TW = T // NW
C = 16
K = 5
NTC = TW // C
NCHUNK = B * NTC
GROUPS = C * N_EMBD // 16


def _body(idx_hbm, tok_hbm, pos_hbm, out_hbm, idx_v, pos0, pos1, *scratch):
    rows = list(scratch[:K])
    sem_g = list(scratch[K:2 * K])
    sem_s = list(scratch[2 * K:3 * K])
    sem_ix = scratch[3 * K]
    sem_pos = [scratch[3 * K + 1], scratch[3 * K + 2]]
    pos = [pos0, pos1]

    wid = lax.axis_index("s") * NC + lax.axis_index("c")
    t0 = wid * TW

    def issue_gather(q):
        tc, b = divmod(q, B)
        return pltpu.async_copy(
            tok_hbm.at[idx_v.at[b, pl.ds(tc * C, C)]], rows[q % K],
            sem_g[q % K])

    def issue_store(q):
        tc, b = divmod(q, B)
        return pltpu.async_copy(
            rows[q % K], out_hbm.at[b, pl.ds(t0 + tc * C, C)], sem_s[q % K])

    def issue_pos(tc):
        return pltpu.async_copy(
            pos_hbm.at[0, pl.ds(t0 + tc * C, C)], pos[tc % 2],
            sem_pos[tc % 2])

    def vpu_add(q):
        tc = q // B
        buf = rows[q % K]
        pbuf = pos[tc % 2]

        @plsc.parallel_loop(0, GROUPS, unroll=8)
        def _(g):
            r = g >> 6
            c = (g & 63) * 16
            plsc.addupdate(buf.at[r, pl.ds(c, 16)], pbuf[r, pl.ds(c, 16)])

    ix = [pltpu.async_copy(idx_hbm.at[b, pl.ds(t0, TW)],
                           idx_v.at[b], sem_ix) for b in range(B)]
    P = [None] * NTC
    P[0] = issue_pos(0)
    P[1] = issue_pos(1)
    for d in ix:
        d.wait()
    G = [None] * NCHUNK
    S = [None] * NCHUNK
    for i in range(K - 2):
        G[i] = issue_gather(i)
    for q in range(NCHUNK):
        tc, b = divmod(q, B)
        G[q].wait()
        if b == 0:
            P[tc].wait()
        if q + K - 2 < NCHUNK:
            if q - 2 >= 0:
                S[q - 2].wait()
            G[q + K - 2] = issue_gather(q + K - 2)
        vpu_add(q)
        S[q] = issue_store(q)
        if b == B - 1 and tc + 2 < NTC:
            P[tc + 2] = issue_pos(tc + 2)
    for q in range(NCHUNK - K, NCHUNK):
        S[q].wait()


_mesh = plsc.VectorSubcoreMesh(core_axis_name="c", subcore_axis_name="s")

_sc_call = functools.partial(
    pl.kernel,
    out_type=jax.ShapeDtypeStruct((B, T, N_EMBD), jnp.float32),
    mesh=_mesh,
    scratch_types=[
        pltpu.VMEM((B, TW), jnp.int32),
        pltpu.VMEM((C, N_EMBD), jnp.float32),
        pltpu.VMEM((C, N_EMBD), jnp.float32),
    ]
    + [pltpu.VMEM((C, N_EMBD), jnp.float32)] * K
    + [pltpu.SemaphoreType.DMA] * (2 * K + 3),
)(_body)


@jax.jit
def kernel(idx, tok_emb, pos_emb):
    return _sc_call(idx.astype(jnp.int32), tok_emb, pos_emb)
